# baseline (speedup 1.0000x reference)
import functools

import jax
import jax.numpy as jnp
from jax import lax
from jax.experimental import pallas as pl
from jax.experimental.pallas import tpu as pltpu
from jax.experimental.pallas import tpu_sc as plsc

N = 1_000_000
D = 16
L = 16
NC, NS = 2, 16
NW = NC * NS

TILE_STRIDE = 31232
CHUNK_ROWS = 1408
GROUPS = CHUNK_ROWS // L
NBUF = 4
SUPERS = 6
CHUNKS = SUPERS * NBUF
MAX_OFF = 999936 - CHUNK_ROWS
TAIL = 128

_mesh = plsc.VectorSubcoreMesh(
    core_axis_name="c", subcore_axis_name="s", num_cores=NC, num_subcores=NS)

F32_INF = float("inf")
I32_MAX = 2**31 - 1


@functools.partial(
    pl.kernel,
    out_type=(
        jax.ShapeDtypeStruct((NW, L), jnp.float32),
        jax.ShapeDtypeStruct((NW, L), jnp.int32),
    ),
    mesh=_mesh,
    scratch_types=[
        pltpu.VMEM((D, CHUNK_ROWS), jnp.float32),
        pltpu.VMEM((D, CHUNK_ROWS), jnp.float32),
        pltpu.VMEM((D, CHUNK_ROWS), jnp.float32),
        pltpu.VMEM((D, CHUNK_ROWS), jnp.float32),
        pltpu.VMEM((L,), jnp.float32),
        pltpu.VMEM((L,), jnp.float32),
        pltpu.VMEM((L,), jnp.int32),
        pltpu.SemaphoreType.DMA,
        pltpu.SemaphoreType.DMA,
        pltpu.SemaphoreType.DMA,
        pltpu.SemaphoreType.DMA,
    ],
    compiler_params=pltpu.CompilerParams(needs_layout_passes=False),
)
def _scan_kernel(q_hbm, keyst_hbm, bd_hbm, bi_hbm, buf0, buf1, buf2, buf3,
                 q_v, od_v, oi_v, sem0, sem1, sem2, sem3):
    wid = lax.axis_index("c") * NS + lax.axis_index("s")
    sems = (sem0, sem1, sem2, sem3)
    bufs = (buf0, buf1, buf2, buf3)

    pltpu.sync_copy(q_hbm, q_v)
    q2 = q_v[...] * 2.0
    qb = [q2.at[jnp.full((L,), d, dtype=jnp.int32)]
          .get(mode="promise_in_bounds") for d in range(D)]
    lane = lax.iota(jnp.int32, L)

    def chunk_off(c):
        return jnp.minimum(wid * TILE_STRIDE + c * CHUNK_ROWS, MAX_OFF)

    def start(b, c):
        pltpu.async_copy(
            keyst_hbm.at[:, pl.ds(chunk_off(c), CHUNK_ROWS)],
            bufs[b], sems[b])

    def wait(b):
        pltpu.make_async_copy(
            keyst_hbm.at[:, pl.ds(0, CHUNK_ROWS)], bufs[b], sems[b]).wait()

    best_d = jnp.full((L,), F32_INF, dtype=jnp.float32)
    best_i = jnp.full((L,), I32_MAX, dtype=jnp.int32)

    for b in range(NBUF):
        start(b, b)

    def super_body(sidx, carry):
        bd, bi = carry
        for b in range(NBUF):
            cn = sidx * NBUF + b
            wait(b)
            off = chunk_off(cn)
            cbuf = bufs[b]

            def gbody(g, carry, cbuf=cbuf):
                bd, bi, row = carry
                j0 = g * (4 * L)
                for u in range(4):
                    j = j0 + u * L
                    ts = [None] * D
                    for d in range(D):
                        v = cbuf[d, pl.ds(j, L)]
                        ts[d] = v * (v - qb[d])
                    while len(ts) > 1:
                        ts = [a + b_ for a, b_ in zip(ts[0::2], ts[1::2])]
                    acc = ts[0]
                    lt = acc < bd
                    bd = jnp.where(lt, acc, bd)
                    bi = jnp.where(lt, row, bi)
                    row = row + L
                return bd, bi, row

            bd, bi, _ = lax.fori_loop(
                0, GROUPS // 4, gbody, (bd, bi, lane + off))

            @pl.when(sidx < SUPERS - 1)
            def _(b=b, cn=cn):
                start(b, cn + NBUF)

        return bd, bi

    best_d, best_i = lax.fori_loop(0, SUPERS, super_body, (best_d, best_i))

    od_v[...] = best_d
    oi_v[...] = best_i
    pltpu.sync_copy(od_v, bd_hbm.at[wid])
    pltpu.sync_copy(oi_v, bi_hbm.at[wid])


def _merge_tc_body(bd_ref, bi_ref, q_ref, tail_ref, valuest_hbm, out_ref,
                   blk_v, sem):
    bd = bd_ref[...]
    bi = bi_ref[...]
    m1 = jnp.min(bd)
    win1 = jnp.min(jnp.where(bd == m1, bi, I32_MAX))
    q_col = q_ref[0].reshape(D, 1)
    tail = tail_ref[...]
    td = jnp.sum(tail * (tail - 2.0 * q_col), axis=0, keepdims=True)
    m2 = jnp.min(td)
    jidx = lax.broadcasted_iota(jnp.int32, (1, TAIL), 1) + (N - TAIL)
    win2 = jnp.min(jnp.where(td == m2, jidx, I32_MAX))
    take2 = (m2 < m1) | ((m2 == m1) & (win2 < win1))
    win = jnp.where(take2, win2, win1)
    base = jnp.minimum((win // 128) * 128, N - 128)
    base = pl.multiple_of(base, 128)
    copy = pltpu.make_async_copy(
        valuest_hbm.at[:, pl.ds(base, 128)], blk_v, sem)
    copy.start()
    copy.wait()
    sel = lax.broadcasted_iota(jnp.int32, (1, 128), 1) == (win - base)
    row = jnp.sum(jnp.where(sel, blk_v[...], 0.0), axis=1)
    out_ref[0, 0] = jnp.sum(row * q_ref[0])


_merge_tc = pl.pallas_call(
    _merge_tc_body,
    out_shape=jax.ShapeDtypeStruct((1, 1), jnp.float32),
    in_specs=[
        pl.BlockSpec(memory_space=pltpu.VMEM),
        pl.BlockSpec(memory_space=pltpu.VMEM),
        pl.BlockSpec(memory_space=pltpu.VMEM),
        pl.BlockSpec(memory_space=pltpu.VMEM),
        pl.BlockSpec(memory_space=pltpu.HBM),
    ],
    out_specs=pl.BlockSpec(memory_space=pltpu.SMEM),
    scratch_shapes=[
        pltpu.VMEM((D, 128), jnp.float32),
        pltpu.SemaphoreType.DMA,
    ],
)


def kernel(query, keys, values):
    keys_t = keys.T
    bd, bi = _scan_kernel(query, keys_t)
    out = _merge_tc(bd, bi, query.reshape(1, D), keys_t[:, N - TAIL:],
                    values.T)
    return out[0]


# speedup vs baseline: 2.3968x; 2.3968x over previous
import functools

import jax
import jax.numpy as jnp
from jax import lax
from jax.experimental import pallas as pl
from jax.experimental.pallas import tpu as pltpu
from jax.experimental.pallas import tpu_sc as plsc

N = 1_000_000
D = 16
L = 16
NC, NS = 2, 16
NW = NC * NS

TILE_STRIDE = 31232
CHUNK_ROWS = 1408
GROUPS = CHUNK_ROWS // L
NBUF = 4
SUPERS = 6
CHUNKS = SUPERS * NBUF
MAX_OFF = 999936 - CHUNK_ROWS
TAIL = 128

_mesh = plsc.VectorSubcoreMesh(
    core_axis_name="c", subcore_axis_name="s", num_cores=NC, num_subcores=NS)

F32_INF = float("inf")
I32_MAX = 2**31 - 1


@functools.partial(
    pl.kernel,
    out_type=(
        jax.ShapeDtypeStruct((NW, L), jnp.float32),
        jax.ShapeDtypeStruct((NW, L), jnp.int32),
    ),
    mesh=_mesh,
    scratch_types=[
        pltpu.VMEM((D, CHUNK_ROWS), jnp.float32),
        pltpu.VMEM((D, CHUNK_ROWS), jnp.float32),
        pltpu.VMEM((D, CHUNK_ROWS), jnp.float32),
        pltpu.VMEM((D, CHUNK_ROWS), jnp.float32),
        pltpu.VMEM((L,), jnp.float32),
        pltpu.VMEM((L,), jnp.float32),
        pltpu.VMEM((L,), jnp.int32),
        pltpu.SemaphoreType.DMA,
        pltpu.SemaphoreType.DMA,
        pltpu.SemaphoreType.DMA,
        pltpu.SemaphoreType.DMA,
    ],
    compiler_params=pltpu.CompilerParams(needs_layout_passes=False),
)
def _scan_kernel(q_hbm, keyst_hbm, bd_hbm, bi_hbm, buf0, buf1, buf2, buf3,
                 q_v, od_v, oi_v, sem0, sem1, sem2, sem3):
    wid = lax.axis_index("c") * NS + lax.axis_index("s")
    sems = (sem0, sem1, sem2, sem3)
    bufs = (buf0, buf1, buf2, buf3)

    pltpu.sync_copy(q_hbm, q_v)
    q2 = q_v[...] * 2.0
    qb = [q2.at[jnp.full((L,), d, dtype=jnp.int32)]
          .get(mode="promise_in_bounds") for d in range(D)]
    lane = lax.iota(jnp.int32, L)

    def chunk_off(c):
        return jnp.minimum(wid * TILE_STRIDE + c * CHUNK_ROWS, MAX_OFF)

    def start(b, c):
        pltpu.async_copy(
            keyst_hbm.at[:, pl.ds(chunk_off(c), CHUNK_ROWS)],
            bufs[b], sems[b])

    def wait(b):
        pltpu.make_async_copy(
            keyst_hbm.at[:, pl.ds(0, CHUNK_ROWS)], bufs[b], sems[b]).wait()

    best_d = jnp.full((L,), F32_INF, dtype=jnp.float32)
    best_i = jnp.full((L,), I32_MAX, dtype=jnp.int32)

    for b in range(NBUF):
        start(b, b)

    def super_body(sidx, carry):
        bd, bi = carry
        for b in range(NBUF):
            cn = sidx * NBUF + b
            wait(b)
            off = chunk_off(cn)
            cbuf = bufs[b]

            def gbody(g, carry, cbuf=cbuf):
                bd, bi, row = carry
                j0 = g * (2 * L)
                for u in range(2):
                    j = j0 + u * L
                    ts = [None] * D
                    for d in range(D):
                        v = cbuf[d, pl.ds(j, L)]
                        ts[d] = v * (v - qb[d])
                    while len(ts) > 1:
                        ts = [a + b_ for a, b_ in zip(ts[0::2], ts[1::2])]
                    acc = ts[0]
                    lt = acc < bd
                    bd = jnp.where(lt, acc, bd)
                    bi = jnp.where(lt, row, bi)
                    row = row + L
                return bd, bi, row

            bd, bi, _ = lax.fori_loop(
                0, GROUPS // 2, gbody, (bd, bi, lane + off))

            @pl.when(sidx < SUPERS - 1)
            def _(b=b, cn=cn):
                start(b, cn + NBUF)

        return bd, bi

    best_d, best_i = lax.fori_loop(0, SUPERS, super_body, (best_d, best_i))

    od_v[...] = best_d
    oi_v[...] = best_i
    pltpu.sync_copy(od_v, bd_hbm.at[wid])
    pltpu.sync_copy(oi_v, bi_hbm.at[wid])


def _merge_tc_body(bd_ref, bi_ref, q_ref, tail_ref, valuest_hbm, out_ref,
                   blk_v, sem):
    bd = bd_ref[...]
    bi = bi_ref[...]
    m1 = jnp.min(bd)
    win1 = jnp.min(jnp.where(bd == m1, bi, I32_MAX))
    q_col = q_ref[0].reshape(D, 1)
    tail = tail_ref[...]
    td = jnp.sum(tail * (tail - 2.0 * q_col), axis=0, keepdims=True)
    m2 = jnp.min(td)
    jidx = lax.broadcasted_iota(jnp.int32, (1, TAIL), 1) + (N - TAIL)
    win2 = jnp.min(jnp.where(td == m2, jidx, I32_MAX))
    take2 = (m2 < m1) | ((m2 == m1) & (win2 < win1))
    win = jnp.where(take2, win2, win1)
    base = jnp.minimum((win // 128) * 128, N - 128)
    base = pl.multiple_of(base, 128)
    copy = pltpu.make_async_copy(
        valuest_hbm.at[:, pl.ds(base, 128)], blk_v, sem)
    copy.start()
    copy.wait()
    sel = lax.broadcasted_iota(jnp.int32, (1, 128), 1) == (win - base)
    row = jnp.sum(jnp.where(sel, blk_v[...], 0.0), axis=1)
    out_ref[0, 0] = jnp.sum(row * q_ref[0])


_merge_tc = pl.pallas_call(
    _merge_tc_body,
    out_shape=jax.ShapeDtypeStruct((1, 1), jnp.float32),
    in_specs=[
        pl.BlockSpec(memory_space=pltpu.VMEM),
        pl.BlockSpec(memory_space=pltpu.VMEM),
        pl.BlockSpec(memory_space=pltpu.VMEM),
        pl.BlockSpec(memory_space=pltpu.VMEM),
        pl.BlockSpec(memory_space=pltpu.HBM),
    ],
    out_specs=pl.BlockSpec(memory_space=pltpu.SMEM),
    scratch_shapes=[
        pltpu.VMEM((D, 128), jnp.float32),
        pltpu.SemaphoreType.DMA,
    ],
)


def kernel(query, keys, values):
    keys_t = keys.T
    bd, bi = _scan_kernel(query, keys_t)
    out = _merge_tc(bd, bi, query.reshape(1, D), keys_t[:, N - TAIL:],
                    values.T)
    return out[0]
